# Initial kernel scaffold; baseline (speedup 1.0000x reference)
import jax
import jax.numpy as jnp
from jax.experimental import pallas as pl


def kernel(x, edge_index, Wl0, Wr0, b0, Wl1, Wr1, b1, Wl2, Wr2, b2, Wl3, Wr3, b3, Wl4, Wr4, b4):
    raise NotImplementedError("write your pallas kernel here")


# speedup vs baseline: 7.7887x; 7.7887x over previous
import functools

import jax
import jax.numpy as jnp
from jax import lax
from jax.experimental import pallas as pl
from jax.experimental.pallas import tpu as pltpu
from jax.experimental.pallas import tpu_sc as plsc

N = 50000
N_PAD = 50176
E = 800000
CHUNK = 128
E_PAD = 802816
N_CHUNKS = E_PAD // CHUNK
IB = 4
N_TILES = 16
ROWS_PER_TILE = N_PAD // N_TILES
BLK = 512
GRID = N_PAD // BLK
F32 = jnp.float32


def _make_agg(width, edge_split):
    mesh = plsc.VectorSubcoreMesh(core_axis_name="c", subcore_axis_name="s")
    chunks_per_tile = (N_CHUNKS // 2 if edge_split else N_CHUNKS) // N_TILES
    nblk = chunks_per_tile // IB

    def body(plo, phi, src, dst, zeros, out_lo, out_hi,
             acc, src_v, dst_v, rows_v, sem):
        c = lax.axis_index("c")
        s = lax.axis_index("s")
        sl = pl.ds(s * ROWS_PER_TILE, ROWS_PER_TILE)
        pltpu.sync_copy(zeros.at[sl], acc.at[sl])
        plsc.subcore_barrier()

        if edge_split:
            base = c * (N_CHUNKS // 2) + s * chunks_per_tile
        else:
            base = s * chunks_per_tile

        def run(p_hbm):
            def blk_body(b, carry):
                blk = base + b * IB
                pltpu.sync_copy(src.at[pl.ds(blk, IB)], src_v)
                pltpu.sync_copy(dst.at[pl.ds(blk, IB)], dst_v)
                descs = [
                    pltpu.async_copy(p_hbm.at[src_v.at[j]], rows_v.at[j], sem)
                    for j in range(IB)
                ]
                for d in descs:
                    d.wait()
                for j in range(IB):
                    pltpu.sync_copy(rows_v.at[j], acc.at[dst_v.at[j]], add=True)
                return carry
            lax.fori_loop(0, nblk, blk_body, 0)

        @pl.when(c == 0)
        def _():
            run(plo)

        @pl.when(c == 1)
        def _():
            run(phi)

        plsc.subcore_barrier()

        @pl.when(c == 0)
        def _():
            pltpu.sync_copy(acc.at[sl], out_lo.at[sl])

        @pl.when(c == 1)
        def _():
            pltpu.sync_copy(acc.at[sl], out_hi.at[sl])

    return pl.kernel(
        body,
        out_type=(jax.ShapeDtypeStruct((N_PAD, width), F32),
                  jax.ShapeDtypeStruct((N_PAD, width), F32)),
        mesh=mesh,
        scratch_types=[
            pltpu.VMEM_SHARED((N_PAD, width), F32),
            pltpu.VMEM((IB, CHUNK), jnp.int32),
            pltpu.VMEM((IB, CHUNK), jnp.int32),
            pltpu.VMEM((IB, CHUNK, width), F32),
            pltpu.SemaphoreType.DMA,
        ],
        compiler_params=pltpu.CompilerParams(use_tc_tiling_on_sc=False),
    )


def _mask_rows(h, i):
    rows = i * BLK + lax.broadcasted_iota(jnp.int32, (BLK, 1), 0)
    return jnp.where(rows < N, h, 0.0)


def _rb(width):
    return pl.BlockSpec((BLK, width), lambda i: (i, 0))


def _full(shape):
    return pl.BlockSpec(shape, lambda i: tuple(0 for _ in shape))


def _dot(a, b):
    return jnp.dot(a, b, preferred_element_type=F32)


def _tc1_body(alo, ahi, xp, wl0, wr0, b0, wl1, wr1, b1,
              p1lo, p1hi, q1, inv_o):
    i = pl.program_id(0)
    s16 = alo[...] + ahi[...]
    inv = 1.0 / jnp.maximum(s16[:, 13:14], 1.0)
    h = _dot(s16 * inv, wl0[...]) + _dot(xp[...], wr0[...]) + b0[...]
    h = _mask_rows(jnp.maximum(h, 0.0), i)
    p1 = _dot(h, wl1[...])
    p1lo[...] = p1[:, :32]
    p1hi[...] = p1[:, 32:]
    q1[...] = _dot(h, wr1[...]) + b1[...]
    inv_o[...] = inv


_tc1 = pl.pallas_call(
    _tc1_body,
    grid=(GRID,),
    in_specs=[_rb(16), _rb(16), _rb(16), _full((16, 64)), _full((16, 64)),
              _full((1, 64)), _full((64, 64)), _full((64, 64)), _full((1, 64))],
    out_specs=[_rb(32), _rb(32), _rb(64), _rb(1)],
    out_shape=[jax.ShapeDtypeStruct((N_PAD, 32), F32),
               jax.ShapeDtypeStruct((N_PAD, 32), F32),
               jax.ShapeDtypeStruct((N_PAD, 64), F32),
               jax.ShapeDtypeStruct((N_PAD, 1), F32)],
)


def _make_tc_mid(wl_cols, split_p):
    def body(alo, ahi, q_in, inv_in, wl, wr, b, *outs):
        i = pl.program_id(0)
        m = jnp.concatenate([alo[...], ahi[...]], axis=1) * inv_in[...]
        h = _mask_rows(jnp.maximum(m + q_in[...], 0.0), i)
        p = _dot(h, wl[...])
        q = _dot(h, wr[...]) + b[...]
        if split_p:
            outs[0][...] = p[:, :32]
            outs[1][...] = p[:, 32:]
            outs[2][...] = q
        else:
            outs[0][...] = p
            outs[1][...] = q

    wr_cols = 64 if split_p else 1
    if split_p:
        out_specs = [_rb(32), _rb(32), _rb(64)]
        out_shape = [jax.ShapeDtypeStruct((N_PAD, 32), F32),
                     jax.ShapeDtypeStruct((N_PAD, 32), F32),
                     jax.ShapeDtypeStruct((N_PAD, 64), F32)]
    else:
        out_specs = [_rb(wl_cols), _rb(1)]
        out_shape = [jax.ShapeDtypeStruct((N_PAD, wl_cols), F32),
                     jax.ShapeDtypeStruct((N_PAD, 1), F32)]
    return pl.pallas_call(
        body,
        grid=(GRID,),
        in_specs=[_rb(32), _rb(32), _rb(64), _rb(1), _full((64, wl_cols)),
                  _full((64, wr_cols)), _full((1, wr_cols))],
        out_specs=out_specs,
        out_shape=out_shape,
    )


_tc_mid = _make_tc_mid(64, True)
_tc_last_proj = _make_tc_mid(16, False)


def _tc5_body(flo, fhi, q4, inv_in, out):
    sm = flo[...][:, :1] + fhi[...][:, :1]
    out[...] = jax.nn.sigmoid(sm * inv_in[...] + q4[...])


_tc5 = pl.pallas_call(
    _tc5_body,
    grid=(GRID,),
    in_specs=[_rb(16), _rb(16), _rb(1), _rb(1)],
    out_specs=_rb(1),
    out_shape=jax.ShapeDtypeStruct((N_PAD, 1), F32),
)

_agg16 = _make_agg(16, edge_split=True)
_agg32 = _make_agg(32, edge_split=False)


def kernel(x, edge_index, Wl0, Wr0, b0, Wl1, Wr1, b1, Wl2, Wr2, b2,
           Wl3, Wr3, b3, Wl4, Wr4, b4):
    src = edge_index[0].astype(jnp.int32)
    dst = edge_index[1].astype(jnp.int32)
    src2 = jnp.concatenate(
        [src, jnp.full((E_PAD - E,), N, jnp.int32)]).reshape(N_CHUNKS, CHUNK)
    dst2 = jnp.concatenate(
        [dst, jnp.zeros((E_PAD - E,), jnp.int32)]).reshape(N_CHUNKS, CHUNK)

    xp = jnp.zeros((N_PAD, 16), F32)
    xp = xp.at[:N, :13].set(x.astype(F32)).at[:N, 13].set(1.0)

    wl0p = jnp.zeros((16, 64), F32).at[:13].set(Wl0)
    wr0p = jnp.zeros((16, 64), F32).at[:13].set(Wr0)
    wl4p = jnp.zeros((64, 16), F32).at[:, :1].set(Wl4)

    z16 = jnp.zeros((N_PAD, 16), F32)
    z32 = jnp.zeros((N_PAD, 32), F32)

    a0lo, a0hi = _agg16(xp, xp, src2, dst2, z16)
    p1lo, p1hi, q1, inv = _tc1(a0lo, a0hi, xp, wl0p, wr0p, b0.reshape(1, 64),
                               Wl1, Wr1, b1.reshape(1, 64))
    a1lo, a1hi = _agg32(p1lo, p1hi, src2, dst2, z32)
    p2lo, p2hi, q2 = _tc_mid(a1lo, a1hi, q1, inv, Wl2, Wr2, b2.reshape(1, 64))
    a2lo, a2hi = _agg32(p2lo, p2hi, src2, dst2, z32)
    p3lo, p3hi, q3 = _tc_mid(a2lo, a2hi, q2, inv, Wl3, Wr3, b3.reshape(1, 64))
    a3lo, a3hi = _agg32(p3lo, p3hi, src2, dst2, z32)
    p4, q4 = _tc_last_proj(a3lo, a3hi, q3, inv, wl4p, Wr4, b4.reshape(1, 1))
    flo, fhi = _agg16(p4, p4, src2, dst2, z16)
    out = _tc5(flo, fhi, q4, inv)
    return out[:N]


# speedup vs baseline: 8.5853x; 1.1023x over previous
import functools

import jax
import jax.numpy as jnp
from jax import lax
from jax.experimental import pallas as pl
from jax.experimental.pallas import tpu as pltpu
from jax.experimental.pallas import tpu_sc as plsc

N = 50000
N_PAD = 50176
E = 800000
CHUNK = 128
E_PAD = 802816
N_CHUNKS = E_PAD // CHUNK
N_TILES = 16
ROWS_PER_TILE = N_PAD // N_TILES
BLK = 512
GRID = N_PAD // BLK
F32 = jnp.float32


def _make_agg(width, edge_split, IB):
    mesh = plsc.VectorSubcoreMesh(core_axis_name="c", subcore_axis_name="s")
    chunks_per_tile = (N_CHUNKS // 2 if edge_split else N_CHUNKS) // N_TILES
    n_it = chunks_per_tile // (2 * IB)

    def body(plo, phi, src, dst, zeros, out_lo, out_hi,
             acc, src_v, dst_v, rows_v, sem_g0, sem_g1, sem_s0, sem_s1):
        c = lax.axis_index("c")
        s = lax.axis_index("s")
        sl = pl.ds(s * ROWS_PER_TILE, ROWS_PER_TILE)
        pltpu.sync_copy(zeros.at[sl], acc.at[sl])
        plsc.subcore_barrier()

        if edge_split:
            base = c * (N_CHUNKS // 2) + s * chunks_per_tile
        else:
            base = s * chunks_per_tile

        sem_g = (sem_g0, sem_g1)
        sem_s = (sem_s0, sem_s1)

        def run(p_hbm):
            def gfire(q, blk):
                pltpu.sync_copy(src.at[pl.ds(blk, IB)], src_v.at[q])
                pltpu.sync_copy(dst.at[pl.ds(blk, IB)], dst_v.at[q])
                for j in range(IB):
                    pltpu.async_copy(p_hbm.at[src_v.at[q].at[j]],
                                     rows_v.at[q].at[j], sem_g[q])

            def gwait(q):
                for j in range(IB):
                    pltpu.make_async_copy(p_hbm.at[src_v.at[q].at[j]],
                                          rows_v.at[q].at[j], sem_g[q]).wait()

            def sfire(q):
                for j in range(IB):
                    pltpu.async_copy(rows_v.at[q].at[j],
                                     acc.at[dst_v.at[q].at[j]], sem_s[q],
                                     add=True)

            def swait(q):
                for j in range(IB):
                    pltpu.make_async_copy(rows_v.at[q].at[j],
                                          acc.at[dst_v.at[q].at[j]],
                                          sem_s[q]).wait()

            def it(i, carry):
                b0 = base + (2 * i) * IB

                @pl.when(i > 0)
                def _():
                    swait(0)
                gfire(0, b0)

                @pl.when(i > 0)
                def _():
                    swait(1)
                gfire(1, b0 + IB)
                gwait(0)
                sfire(0)
                gwait(1)
                sfire(1)
                return carry

            lax.fori_loop(0, n_it, it, 0)
            swait(0)
            swait(1)

        @pl.when(c == 0)
        def _():
            run(plo)

        @pl.when(c == 1)
        def _():
            run(phi)

        plsc.subcore_barrier()

        @pl.when(c == 0)
        def _():
            pltpu.sync_copy(acc.at[sl], out_lo.at[sl])

        @pl.when(c == 1)
        def _():
            pltpu.sync_copy(acc.at[sl], out_hi.at[sl])

    return pl.kernel(
        body,
        out_type=(jax.ShapeDtypeStruct((N_PAD, width), F32),
                  jax.ShapeDtypeStruct((N_PAD, width), F32)),
        mesh=mesh,
        scratch_types=[
            pltpu.VMEM_SHARED((N_PAD, width), F32),
            pltpu.VMEM((2, IB, CHUNK), jnp.int32),
            pltpu.VMEM((2, IB, CHUNK), jnp.int32),
            pltpu.VMEM((2, IB, CHUNK, width), F32),
            pltpu.SemaphoreType.DMA,
            pltpu.SemaphoreType.DMA,
            pltpu.SemaphoreType.DMA,
            pltpu.SemaphoreType.DMA,
        ],
        compiler_params=pltpu.CompilerParams(use_tc_tiling_on_sc=False),
    )


def _mask_rows(h, i):
    rows = i * BLK + lax.broadcasted_iota(jnp.int32, (BLK, 1), 0)
    return jnp.where(rows < N, h, 0.0)


def _rb(width):
    return pl.BlockSpec((BLK, width), lambda i: (i, 0))


def _full(shape):
    return pl.BlockSpec(shape, lambda i: tuple(0 for _ in shape))


def _dot(a, b):
    return jnp.dot(a, b, preferred_element_type=F32)


def _tc1_body(alo, ahi, xp, wl0, wr0, b0, wl1, wr1, b1,
              p1lo, p1hi, q1, inv_o):
    i = pl.program_id(0)
    s16 = alo[...] + ahi[...]
    inv = 1.0 / jnp.maximum(s16[:, 13:14], 1.0)
    h = _dot(s16 * inv, wl0[...]) + _dot(xp[...], wr0[...]) + b0[...]
    h = _mask_rows(jnp.maximum(h, 0.0), i)
    p1 = _dot(h, wl1[...])
    p1lo[...] = p1[:, :32]
    p1hi[...] = p1[:, 32:]
    q1[...] = _dot(h, wr1[...]) + b1[...]
    inv_o[...] = inv


_tc1 = pl.pallas_call(
    _tc1_body,
    grid=(GRID,),
    in_specs=[_rb(16), _rb(16), _rb(16), _full((16, 64)), _full((16, 64)),
              _full((1, 64)), _full((64, 64)), _full((64, 64)), _full((1, 64))],
    out_specs=[_rb(32), _rb(32), _rb(64), _rb(1)],
    out_shape=[jax.ShapeDtypeStruct((N_PAD, 32), F32),
               jax.ShapeDtypeStruct((N_PAD, 32), F32),
               jax.ShapeDtypeStruct((N_PAD, 64), F32),
               jax.ShapeDtypeStruct((N_PAD, 1), F32)],
)


def _make_tc_mid(wl_cols, split_p):
    def body(alo, ahi, q_in, inv_in, wl, wr, b, *outs):
        i = pl.program_id(0)
        m = jnp.concatenate([alo[...], ahi[...]], axis=1) * inv_in[...]
        h = _mask_rows(jnp.maximum(m + q_in[...], 0.0), i)
        p = _dot(h, wl[...])
        q = _dot(h, wr[...]) + b[...]
        if split_p:
            outs[0][...] = p[:, :32]
            outs[1][...] = p[:, 32:]
            outs[2][...] = q
        else:
            outs[0][...] = p
            outs[1][...] = q

    wr_cols = 64 if split_p else 1
    if split_p:
        out_specs = [_rb(32), _rb(32), _rb(64)]
        out_shape = [jax.ShapeDtypeStruct((N_PAD, 32), F32),
                     jax.ShapeDtypeStruct((N_PAD, 32), F32),
                     jax.ShapeDtypeStruct((N_PAD, 64), F32)]
    else:
        out_specs = [_rb(wl_cols), _rb(1)]
        out_shape = [jax.ShapeDtypeStruct((N_PAD, wl_cols), F32),
                     jax.ShapeDtypeStruct((N_PAD, 1), F32)]
    return pl.pallas_call(
        body,
        grid=(GRID,),
        in_specs=[_rb(32), _rb(32), _rb(64), _rb(1), _full((64, wl_cols)),
                  _full((64, wr_cols)), _full((1, wr_cols))],
        out_specs=out_specs,
        out_shape=out_shape,
    )


_tc_mid = _make_tc_mid(64, True)
_tc_last_proj = _make_tc_mid(16, False)


def _tc5_body(flo, fhi, q4, inv_in, out):
    sm = flo[...][:, :1] + fhi[...][:, :1]
    out[...] = jax.nn.sigmoid(sm * inv_in[...] + q4[...])


_tc5 = pl.pallas_call(
    _tc5_body,
    grid=(GRID,),
    in_specs=[_rb(16), _rb(16), _rb(1), _rb(1)],
    out_specs=_rb(1),
    out_shape=jax.ShapeDtypeStruct((N_PAD, 1), F32),
)

_agg16 = _make_agg(16, edge_split=True, IB=7)
_agg32 = _make_agg(32, edge_split=False, IB=2)


def kernel(x, edge_index, Wl0, Wr0, b0, Wl1, Wr1, b1, Wl2, Wr2, b2,
           Wl3, Wr3, b3, Wl4, Wr4, b4):
    src = edge_index[0].astype(jnp.int32)
    dst = edge_index[1].astype(jnp.int32)
    src2 = jnp.concatenate(
        [src, jnp.full((E_PAD - E,), N, jnp.int32)]).reshape(N_CHUNKS, CHUNK)
    dst2 = jnp.concatenate(
        [dst, jnp.zeros((E_PAD - E,), jnp.int32)]).reshape(N_CHUNKS, CHUNK)

    xp = jnp.zeros((N_PAD, 16), F32)
    xp = xp.at[:N, :13].set(x.astype(F32)).at[:N, 13].set(1.0)

    wl0p = jnp.zeros((16, 64), F32).at[:13].set(Wl0)
    wr0p = jnp.zeros((16, 64), F32).at[:13].set(Wr0)
    wl4p = jnp.zeros((64, 16), F32).at[:, :1].set(Wl4)

    z16 = jnp.zeros((N_PAD, 16), F32)
    z32 = jnp.zeros((N_PAD, 32), F32)

    a0lo, a0hi = _agg16(xp, xp, src2, dst2, z16)
    p1lo, p1hi, q1, inv = _tc1(a0lo, a0hi, xp, wl0p, wr0p, b0.reshape(1, 64),
                               Wl1, Wr1, b1.reshape(1, 64))
    a1lo, a1hi = _agg32(p1lo, p1hi, src2, dst2, z32)
    p2lo, p2hi, q2 = _tc_mid(a1lo, a1hi, q1, inv, Wl2, Wr2, b2.reshape(1, 64))
    a2lo, a2hi = _agg32(p2lo, p2hi, src2, dst2, z32)
    p3lo, p3hi, q3 = _tc_mid(a2lo, a2hi, q2, inv, Wl3, Wr3, b3.reshape(1, 64))
    a3lo, a3hi = _agg32(p3lo, p3hi, src2, dst2, z32)
    p4, q4 = _tc_last_proj(a3lo, a3hi, q3, inv, wl4p, Wr4, b4.reshape(1, 1))
    flo, fhi = _agg16(p4, p4, src2, dst2, z16)
    out = _tc5(flo, fhi, q4, inv)
    return out[:N]


# speedup vs baseline: 10.1756x; 1.1852x over previous
import functools

import jax
import jax.numpy as jnp
from jax import lax
from jax.experimental import pallas as pl
from jax.experimental.pallas import tpu as pltpu
from jax.experimental.pallas import tpu_sc as plsc

N = 50000
N_PAD = 50176
E = 800000
CHUNK = 128
E_PAD = 802816
N_CHUNKS = E_PAD // CHUNK
N_TILES = 16
ROWS_PER_TILE = N_PAD // N_TILES
BLK = 3136
GRID = N_PAD // BLK
F32 = jnp.float32


def _make_agg(width, edge_split, IB):
    mesh = plsc.VectorSubcoreMesh(core_axis_name="c", subcore_axis_name="s")
    chunks_per_tile = (N_CHUNKS // 2 if edge_split else N_CHUNKS) // N_TILES
    n_it = chunks_per_tile // (2 * IB)

    def body(plo, phi, src, dst, zeros, out_lo, out_hi,
             acc, src_v, dst_v, rows_v, sem_g0, sem_g1, sem_s0, sem_s1):
        c = lax.axis_index("c")
        s = lax.axis_index("s")
        sl = pl.ds(s * ROWS_PER_TILE, ROWS_PER_TILE)
        pltpu.sync_copy(zeros.at[sl], acc.at[sl])
        plsc.subcore_barrier()

        if edge_split:
            base = c * (N_CHUNKS // 2) + s * chunks_per_tile
        else:
            base = s * chunks_per_tile

        sem_g = (sem_g0, sem_g1)
        sem_s = (sem_s0, sem_s1)

        def run(p_hbm):
            def gfire(q, blk):
                pltpu.sync_copy(src.at[pl.ds(blk, IB)], src_v.at[q])
                pltpu.sync_copy(dst.at[pl.ds(blk, IB)], dst_v.at[q])
                for j in range(IB):
                    pltpu.async_copy(p_hbm.at[src_v.at[q].at[j]],
                                     rows_v.at[q].at[j], sem_g[q])

            def gwait(q):
                for j in range(IB):
                    pltpu.make_async_copy(p_hbm.at[src_v.at[q].at[j]],
                                          rows_v.at[q].at[j], sem_g[q]).wait()

            def sfire(q):
                for j in range(IB):
                    pltpu.async_copy(rows_v.at[q].at[j],
                                     acc.at[dst_v.at[q].at[j]], sem_s[q],
                                     add=True)

            def swait(q):
                for j in range(IB):
                    pltpu.make_async_copy(rows_v.at[q].at[j],
                                          acc.at[dst_v.at[q].at[j]],
                                          sem_s[q]).wait()

            def it(i, carry):
                b0 = base + (2 * i) * IB

                @pl.when(i > 0)
                def _():
                    swait(0)
                gfire(0, b0)

                @pl.when(i > 0)
                def _():
                    swait(1)
                gfire(1, b0 + IB)
                gwait(0)
                sfire(0)
                gwait(1)
                sfire(1)
                return carry

            lax.fori_loop(0, n_it, it, 0)
            swait(0)
            swait(1)

        @pl.when(c == 0)
        def _():
            run(plo)

        @pl.when(c == 1)
        def _():
            run(phi)

        plsc.subcore_barrier()

        @pl.when(c == 0)
        def _():
            pltpu.sync_copy(acc.at[sl], out_lo.at[sl])

        @pl.when(c == 1)
        def _():
            pltpu.sync_copy(acc.at[sl], out_hi.at[sl])

    return pl.kernel(
        body,
        out_type=(jax.ShapeDtypeStruct((N_PAD, width), F32),
                  jax.ShapeDtypeStruct((N_PAD, width), F32)),
        mesh=mesh,
        scratch_types=[
            pltpu.VMEM_SHARED((N_PAD, width), F32),
            pltpu.VMEM((2, IB, CHUNK), jnp.int32),
            pltpu.VMEM((2, IB, CHUNK), jnp.int32),
            pltpu.VMEM((2, IB, CHUNK, width), F32),
            pltpu.SemaphoreType.DMA,
            pltpu.SemaphoreType.DMA,
            pltpu.SemaphoreType.DMA,
            pltpu.SemaphoreType.DMA,
        ],
        compiler_params=pltpu.CompilerParams(use_tc_tiling_on_sc=False),
    )


def _mask_rows(h, i):
    rows = i * BLK + lax.broadcasted_iota(jnp.int32, (BLK, 1), 0)
    return jnp.where(rows < N, h, 0.0)


def _rb(width):
    return pl.BlockSpec((BLK, width), lambda i: (i, 0))


def _full(shape):
    return pl.BlockSpec(shape, lambda i: tuple(0 for _ in shape))


def _dot(a, b):
    return jnp.dot(a, b, preferred_element_type=F32)


def _tc0_body(x_ref, xp_ref):
    i = pl.program_id(0)
    rows = i * BLK + lax.broadcasted_iota(jnp.int32, (BLK, 1), 0)
    mask = rows < N
    xb = jnp.where(mask, x_ref[...], 0.0)
    ones = jnp.where(mask, 1.0, 0.0).astype(F32)
    xp_ref[...] = jnp.concatenate([xb, ones, jnp.zeros((BLK, 2), F32)], axis=1)


_tc0 = pl.pallas_call(
    _tc0_body,
    grid=(GRID,),
    in_specs=[pl.BlockSpec((BLK, 13), lambda i: (i, 0))],
    out_specs=pl.BlockSpec((BLK, 16), lambda i: (i, 0)),
    out_shape=jax.ShapeDtypeStruct((N_PAD, 16), F32),
)


def _tc1_body(alo, ahi, xp, wl0, wr0, b0, wl1, wr1, b1,
              p1lo, p1hi, q1, inv_o):
    i = pl.program_id(0)
    s16 = alo[...] + ahi[...]
    inv = 1.0 / jnp.maximum(s16[:, 13:14], 1.0)
    h = _dot(s16 * inv, wl0[...]) + _dot(xp[...], wr0[...]) + b0[...]
    h = _mask_rows(jnp.maximum(h, 0.0), i)
    p1 = _dot(h, wl1[...])
    p1lo[...] = p1[:, :32]
    p1hi[...] = p1[:, 32:]
    q1[...] = _dot(h, wr1[...]) + b1[...]
    inv_o[...] = inv


_tc1 = pl.pallas_call(
    _tc1_body,
    grid=(GRID,),
    in_specs=[_rb(16), _rb(16), _rb(16), _full((16, 64)), _full((16, 64)),
              _full((1, 64)), _full((64, 64)), _full((64, 64)), _full((1, 64))],
    out_specs=[_rb(32), _rb(32), _rb(64), _rb(1)],
    out_shape=[jax.ShapeDtypeStruct((N_PAD, 32), F32),
               jax.ShapeDtypeStruct((N_PAD, 32), F32),
               jax.ShapeDtypeStruct((N_PAD, 64), F32),
               jax.ShapeDtypeStruct((N_PAD, 1), F32)],
)


def _make_tc_mid(wl_cols, split_p):
    def body(alo, ahi, q_in, inv_in, wl, wr, b, *outs):
        i = pl.program_id(0)
        m = jnp.concatenate([alo[...], ahi[...]], axis=1) * inv_in[...]
        h = _mask_rows(jnp.maximum(m + q_in[...], 0.0), i)
        p = _dot(h, wl[...])
        q = _dot(h, wr[...]) + b[...]
        if split_p:
            outs[0][...] = p[:, :32]
            outs[1][...] = p[:, 32:]
            outs[2][...] = q
        else:
            outs[0][...] = p
            outs[1][...] = q

    wr_cols = 64 if split_p else 1
    if split_p:
        out_specs = [_rb(32), _rb(32), _rb(64)]
        out_shape = [jax.ShapeDtypeStruct((N_PAD, 32), F32),
                     jax.ShapeDtypeStruct((N_PAD, 32), F32),
                     jax.ShapeDtypeStruct((N_PAD, 64), F32)]
    else:
        out_specs = [_rb(wl_cols), _rb(1)]
        out_shape = [jax.ShapeDtypeStruct((N_PAD, wl_cols), F32),
                     jax.ShapeDtypeStruct((N_PAD, 1), F32)]
    return pl.pallas_call(
        body,
        grid=(GRID,),
        in_specs=[_rb(32), _rb(32), _rb(64), _rb(1), _full((64, wl_cols)),
                  _full((64, wr_cols)), _full((1, wr_cols))],
        out_specs=out_specs,
        out_shape=out_shape,
    )


_tc_mid = _make_tc_mid(64, True)
_tc_last_proj = _make_tc_mid(16, False)


def _tc5_body(flo, fhi, q4, inv_in, out):
    sm = flo[...][:, :1] + fhi[...][:, :1]
    out[...] = jax.nn.sigmoid(sm * inv_in[...] + q4[...])


_tc5 = pl.pallas_call(
    _tc5_body,
    grid=(GRID,),
    in_specs=[_rb(16), _rb(16), _rb(1), _rb(1)],
    out_specs=_rb(1),
    out_shape=jax.ShapeDtypeStruct((N_PAD, 1), F32),
)

_agg16 = _make_agg(16, edge_split=True, IB=7)
_agg32 = _make_agg(32, edge_split=False, IB=2)


def kernel(x, edge_index, Wl0, Wr0, b0, Wl1, Wr1, b1, Wl2, Wr2, b2,
           Wl3, Wr3, b3, Wl4, Wr4, b4):
    src = edge_index[0].astype(jnp.int32)
    dst = edge_index[1].astype(jnp.int32)
    src2 = jnp.concatenate(
        [src, jnp.full((E_PAD - E,), N, jnp.int32)]).reshape(N_CHUNKS, CHUNK)
    dst2 = jnp.concatenate(
        [dst, jnp.zeros((E_PAD - E,), jnp.int32)]).reshape(N_CHUNKS, CHUNK)

    xp = _tc0(x.astype(F32))

    wl0p = jnp.zeros((16, 64), F32).at[:13].set(Wl0)
    wr0p = jnp.zeros((16, 64), F32).at[:13].set(Wr0)
    wl4p = jnp.zeros((64, 16), F32).at[:, :1].set(Wl4)

    z16 = jnp.zeros((N_PAD, 16), F32)
    z32 = jnp.zeros((N_PAD, 32), F32)

    a0lo, a0hi = _agg16(xp, xp, src2, dst2, z16)
    p1lo, p1hi, q1, inv = _tc1(a0lo, a0hi, xp, wl0p, wr0p, b0.reshape(1, 64),
                               Wl1, Wr1, b1.reshape(1, 64))
    a1lo, a1hi = _agg32(p1lo, p1hi, src2, dst2, z32)
    p2lo, p2hi, q2 = _tc_mid(a1lo, a1hi, q1, inv, Wl2, Wr2, b2.reshape(1, 64))
    a2lo, a2hi = _agg32(p2lo, p2hi, src2, dst2, z32)
    p3lo, p3hi, q3 = _tc_mid(a2lo, a2hi, q2, inv, Wl3, Wr3, b3.reshape(1, 64))
    a3lo, a3hi = _agg32(p3lo, p3hi, src2, dst2, z32)
    p4, q4 = _tc_last_proj(a3lo, a3hi, q3, inv, wl4p, Wr4, b4.reshape(1, 1))
    flo, fhi = _agg16(p4, p4, src2, dst2, z16)
    out = _tc5(flo, fhi, q4, inv)
    return out[:N]


# speedup vs baseline: 10.3919x; 1.0213x over previous
import functools

import jax
import jax.numpy as jnp
from jax import lax
from jax.experimental import pallas as pl
from jax.experimental.pallas import tpu as pltpu
from jax.experimental.pallas import tpu_sc as plsc

N = 50000
N_PAD = 50176
E = 800000
CHUNK = 128
E_PAD = 802816
N_CHUNKS = E_PAD // CHUNK
N_TILES = 16
ROWS_PER_TILE = N_PAD // N_TILES
BLK = 3136
GRID = N_PAD // BLK
F32 = jnp.float32


def _make_agg(width, edge_split, IB):
    mesh = plsc.VectorSubcoreMesh(core_axis_name="c", subcore_axis_name="s")
    chunks_per_tile = (N_CHUNKS // 2 if edge_split else N_CHUNKS) // N_TILES
    n_it = chunks_per_tile // (2 * IB)

    def body(plo, phi, src, dst, zeros, out_lo, out_hi,
             acc, src_v, dst_v, rows_v, sem_g0, sem_g1, sem_s0, sem_s1):
        c = lax.axis_index("c")
        s = lax.axis_index("s")
        sl = pl.ds(s * ROWS_PER_TILE, ROWS_PER_TILE)
        pltpu.sync_copy(zeros.at[sl], acc.at[sl])
        plsc.subcore_barrier()

        if edge_split:
            base = c * (N_CHUNKS // 2) + s * chunks_per_tile
        else:
            base = s * chunks_per_tile

        sem_g = (sem_g0, sem_g1)
        sem_s = (sem_s0, sem_s1)

        def run(p_hbm):
            def gfire(q, blk):
                pltpu.sync_copy(src.at[pl.ds(blk, IB)], src_v.at[q])
                pltpu.sync_copy(dst.at[pl.ds(blk, IB)], dst_v.at[q])
                for j in range(IB):
                    pltpu.async_copy(p_hbm.at[src_v.at[q].at[j]],
                                     rows_v.at[q].at[j], sem_g[q])

            def gwait(q):
                for j in range(IB):
                    pltpu.make_async_copy(p_hbm.at[src_v.at[q].at[j]],
                                          rows_v.at[q].at[j], sem_g[q]).wait()

            def sfire(q):
                for j in range(IB):
                    pltpu.async_copy(rows_v.at[q].at[j],
                                     acc.at[dst_v.at[q].at[j]], sem_s[q],
                                     add=True)

            def swait(q):
                for j in range(IB):
                    pltpu.make_async_copy(rows_v.at[q].at[j],
                                          acc.at[dst_v.at[q].at[j]],
                                          sem_s[q]).wait()

            def it(i, carry):
                b0 = base + (2 * i) * IB

                @pl.when(i > 0)
                def _():
                    swait(0)
                gfire(0, b0)

                @pl.when(i > 0)
                def _():
                    swait(1)
                gfire(1, b0 + IB)
                gwait(0)
                sfire(0)
                gwait(1)
                sfire(1)
                return carry

            lax.fori_loop(0, n_it, it, 0)
            swait(0)
            swait(1)

        @pl.when(c == 0)
        def _():
            run(plo)

        @pl.when(c == 1)
        def _():
            run(phi)

        plsc.subcore_barrier()

        @pl.when(c == 0)
        def _():
            pltpu.sync_copy(acc.at[sl], out_lo.at[sl])

        @pl.when(c == 1)
        def _():
            pltpu.sync_copy(acc.at[sl], out_hi.at[sl])

    return pl.kernel(
        body,
        out_type=(jax.ShapeDtypeStruct((N_PAD, width), F32),
                  jax.ShapeDtypeStruct((N_PAD, width), F32)),
        mesh=mesh,
        scratch_types=[
            pltpu.VMEM_SHARED((N_PAD, width), F32),
            pltpu.VMEM((2, IB, CHUNK), jnp.int32),
            pltpu.VMEM((2, IB, CHUNK), jnp.int32),
            pltpu.VMEM((2, IB, CHUNK, width), F32),
            pltpu.SemaphoreType.DMA,
            pltpu.SemaphoreType.DMA,
            pltpu.SemaphoreType.DMA,
            pltpu.SemaphoreType.DMA,
        ],
        compiler_params=pltpu.CompilerParams(use_tc_tiling_on_sc=False),
    )


def _make_agg1(IB):
    mesh = plsc.VectorSubcoreMesh(core_axis_name="c", subcore_axis_name="s")
    chunks_per_tile = (N_CHUNKS // 2) // N_TILES
    n_it = chunks_per_tile // (2 * IB)

    def body(p_hbm, src, dst, zeros, out_lo, out_hi,
             acc, src_v, dst_v, rows_v, sem_g0, sem_g1, sem_s0, sem_s1):
        c = lax.axis_index("c")
        s = lax.axis_index("s")
        sl = pl.ds(s * ROWS_PER_TILE, ROWS_PER_TILE)
        pltpu.sync_copy(zeros.at[sl], acc.at[sl])
        plsc.subcore_barrier()

        base = c * (N_CHUNKS // 2) + s * chunks_per_tile
        sem_g = (sem_g0, sem_g1)
        sem_s = (sem_s0, sem_s1)

        def gfire(q, blk):
            pltpu.sync_copy(src.at[pl.ds(blk, IB)], src_v.at[q])
            pltpu.sync_copy(dst.at[pl.ds(blk, IB)], dst_v.at[q])
            for j in range(IB):
                pltpu.async_copy(p_hbm.at[src_v.at[q].at[j]],
                                 rows_v.at[q].at[j], sem_g[q])

        def gwait(q):
            for j in range(IB):
                pltpu.make_async_copy(p_hbm.at[src_v.at[q].at[j]],
                                      rows_v.at[q].at[j], sem_g[q]).wait()

        def sfire(q):
            for j in range(IB):
                pltpu.async_copy(rows_v.at[q].at[j],
                                 acc.at[dst_v.at[q].at[j]], sem_s[q], add=True)

        def swait(q):
            for j in range(IB):
                pltpu.make_async_copy(rows_v.at[q].at[j],
                                      acc.at[dst_v.at[q].at[j]],
                                      sem_s[q]).wait()

        def it(i, carry):
            b0 = base + (2 * i) * IB

            @pl.when(i > 0)
            def _():
                swait(0)
            gfire(0, b0)

            @pl.when(i > 0)
            def _():
                swait(1)
            gfire(1, b0 + IB)
            gwait(0)
            sfire(0)
            gwait(1)
            sfire(1)
            return carry

        lax.fori_loop(0, n_it, it, 0)
        swait(0)
        swait(1)

        plsc.subcore_barrier()

        @pl.when(c == 0)
        def _():
            pltpu.sync_copy(acc.at[sl], out_lo.at[sl])

        @pl.when(c == 1)
        def _():
            pltpu.sync_copy(acc.at[sl], out_hi.at[sl])

    return pl.kernel(
        body,
        out_type=(jax.ShapeDtypeStruct((N_PAD,), F32),
                  jax.ShapeDtypeStruct((N_PAD,), F32)),
        mesh=mesh,
        scratch_types=[
            pltpu.VMEM_SHARED((N_PAD,), F32),
            pltpu.VMEM((2, IB, CHUNK), jnp.int32),
            pltpu.VMEM((2, IB, CHUNK), jnp.int32),
            pltpu.VMEM((2, IB, CHUNK), F32),
            pltpu.SemaphoreType.DMA,
            pltpu.SemaphoreType.DMA,
            pltpu.SemaphoreType.DMA,
            pltpu.SemaphoreType.DMA,
        ],
        compiler_params=pltpu.CompilerParams(use_tc_tiling_on_sc=False),
    )


def _mask_rows(h, i, blk=None):
    blk = BLK if blk is None else blk
    rows = i * blk + lax.broadcasted_iota(jnp.int32, (blk, 1), 0)
    return jnp.where(rows < N, h, 0.0)


def _rb(width):
    return pl.BlockSpec((BLK, width), lambda i: (i, 0))


def _full(shape):
    return pl.BlockSpec(shape, lambda i: tuple(0 for _ in shape))


def _dot(a, b):
    return jnp.dot(a, b, preferred_element_type=F32)


def _tc0_body(x_ref, xp_ref):
    i = pl.program_id(0)
    rows = i * BLK + lax.broadcasted_iota(jnp.int32, (BLK, 1), 0)
    mask = rows < N
    xb = jnp.where(mask, x_ref[...], 0.0)
    ones = jnp.where(mask, 1.0, 0.0).astype(F32)
    xp_ref[...] = jnp.concatenate([xb, ones, jnp.zeros((BLK, 2), F32)], axis=1)


_tc0 = pl.pallas_call(
    _tc0_body,
    grid=(GRID,),
    in_specs=[pl.BlockSpec((BLK, 13), lambda i: (i, 0))],
    out_specs=pl.BlockSpec((BLK, 16), lambda i: (i, 0)),
    out_shape=jax.ShapeDtypeStruct((N_PAD, 16), F32),
)


def _tc1_body(alo, ahi, xp, wl0, wr0, b0, wl1, wr1, b1,
              p1lo, p1hi, q1, inv_o):
    i = pl.program_id(0)
    s16 = alo[...] + ahi[...]
    inv = 1.0 / jnp.maximum(s16[:, 13:14], 1.0)
    h = _dot(s16 * inv, wl0[...]) + _dot(xp[...], wr0[...]) + b0[...]
    h = _mask_rows(jnp.maximum(h, 0.0), i)
    p1 = _dot(h, wl1[...])
    p1lo[...] = p1[:, :32]
    p1hi[...] = p1[:, 32:]
    q1[...] = _dot(h, wr1[...]) + b1[...]
    inv_o[...] = inv


_tc1 = pl.pallas_call(
    _tc1_body,
    grid=(GRID,),
    in_specs=[_rb(16), _rb(16), _rb(16), _full((16, 64)), _full((16, 64)),
              _full((1, 64)), _full((64, 64)), _full((64, 64)), _full((1, 64))],
    out_specs=[_rb(32), _rb(32), _rb(64), _rb(1)],
    out_shape=[jax.ShapeDtypeStruct((N_PAD, 32), F32),
               jax.ShapeDtypeStruct((N_PAD, 32), F32),
               jax.ShapeDtypeStruct((N_PAD, 64), F32),
               jax.ShapeDtypeStruct((N_PAD, 1), F32)],
)


def _make_tc_mid(wl_cols, split_p, blk=BLK):
    def body(alo, ahi, q_in, inv_in, wl, wr, b, *outs):
        i = pl.program_id(0)
        m = jnp.concatenate([alo[...], ahi[...]], axis=1) * inv_in[...]
        h = _mask_rows(jnp.maximum(m + q_in[...], 0.0), i, blk)
        p = _dot(h, wl[...])
        q = _dot(h, wr[...]) + b[...]
        if split_p:
            outs[0][...] = p[:, :32]
            outs[1][...] = p[:, 32:]
            outs[2][...] = q
        else:
            outs[0][...] = p[:, 0]
            outs[1][...] = q

    wr_cols = 64 if split_p else 1
    rb = lambda w: pl.BlockSpec((blk, w), lambda i: (i, 0))
    if split_p:
        out_specs = [rb(32), rb(32), rb(64)]
        out_shape = [jax.ShapeDtypeStruct((N_PAD, 32), F32),
                     jax.ShapeDtypeStruct((N_PAD, 32), F32),
                     jax.ShapeDtypeStruct((N_PAD, 64), F32)]
    else:
        out_specs = [pl.BlockSpec((blk,), lambda i: (i,)), rb(1)]
        out_shape = [jax.ShapeDtypeStruct((N_PAD,), F32),
                     jax.ShapeDtypeStruct((N_PAD, 1), F32)]
    return pl.pallas_call(
        body,
        grid=(N_PAD // blk,),
        in_specs=[rb(32), rb(32), rb(64), rb(1), _full((64, wl_cols)),
                  _full((64, wr_cols)), _full((1, wr_cols))],
        out_specs=out_specs,
        out_shape=out_shape,
    )


_tc_mid = _make_tc_mid(64, True)
_tc_last_proj = _make_tc_mid(1, False, blk=1024)


BLK5 = 1024


def _tc5_body(flo, fhi, q4, inv_in, out):
    sm = jnp.reshape(flo[...] + fhi[...], (BLK5, 1))
    out[...] = jax.nn.sigmoid(sm * inv_in[...] + q4[...])


_tc5 = pl.pallas_call(
    _tc5_body,
    grid=(N_PAD // BLK5,),
    in_specs=[pl.BlockSpec((BLK5,), lambda i: (i,)),
              pl.BlockSpec((BLK5,), lambda i: (i,)),
              pl.BlockSpec((BLK5, 1), lambda i: (i, 0)),
              pl.BlockSpec((BLK5, 1), lambda i: (i, 0))],
    out_specs=pl.BlockSpec((BLK5, 1), lambda i: (i, 0)),
    out_shape=jax.ShapeDtypeStruct((N_PAD, 1), F32),
)

_agg16 = _make_agg(16, edge_split=True, IB=7)
_agg32 = _make_agg(32, edge_split=False, IB=2)
_agg1 = _make_agg1(IB=7)


def kernel(x, edge_index, Wl0, Wr0, b0, Wl1, Wr1, b1, Wl2, Wr2, b2,
           Wl3, Wr3, b3, Wl4, Wr4, b4):
    src = edge_index[0].astype(jnp.int32)
    dst = edge_index[1].astype(jnp.int32)
    src2 = jnp.concatenate(
        [src, jnp.full((E_PAD - E,), N, jnp.int32)]).reshape(N_CHUNKS, CHUNK)
    dst2 = jnp.concatenate(
        [dst, jnp.zeros((E_PAD - E,), jnp.int32)]).reshape(N_CHUNKS, CHUNK)

    xp = _tc0(x.astype(F32))

    wl0p = jnp.zeros((16, 64), F32).at[:13].set(Wl0)
    wr0p = jnp.zeros((16, 64), F32).at[:13].set(Wr0)

    z16 = jnp.zeros((N_PAD, 16), F32)
    z32 = jnp.zeros((N_PAD, 32), F32)
    z1 = jnp.zeros((N_PAD,), F32)

    a0lo, a0hi = _agg16(xp, xp, src2, dst2, z16)
    p1lo, p1hi, q1, inv = _tc1(a0lo, a0hi, xp, wl0p, wr0p, b0.reshape(1, 64),
                               Wl1, Wr1, b1.reshape(1, 64))
    a1lo, a1hi = _agg32(p1lo, p1hi, src2, dst2, z32)
    p2lo, p2hi, q2 = _tc_mid(a1lo, a1hi, q1, inv, Wl2, Wr2, b2.reshape(1, 64))
    a2lo, a2hi = _agg32(p2lo, p2hi, src2, dst2, z32)
    p3lo, p3hi, q3 = _tc_mid(a2lo, a2hi, q2, inv, Wl3, Wr3, b3.reshape(1, 64))
    a3lo, a3hi = _agg32(p3lo, p3hi, src2, dst2, z32)
    p4, q4 = _tc_last_proj(a3lo, a3hi, q3, inv, Wl4, Wr4, b4.reshape(1, 1))
    flo, fhi = _agg1(p4, src2, dst2, z1)
    out = _tc5(flo, fhi, q4, inv)
    return out[:N]
